# baseline (speedup 1.0000x reference)
import functools

import jax
import jax.numpy as jnp
from jax import lax
from jax.experimental import pallas as pl
from jax.experimental.pallas import tpu as pltpu
from jax.experimental.pallas import tpu_sc as plsc

N = 320000
C = 10000
DF = 128
DZ = 128
CHUNK = 128
NCHUNK = N // CHUNK
NC, NS = 2, 16
NW = NC * NS
PW = NCHUNK // NW
TAIL = NCHUNK - PW * NW
CS = 624
CSREM = C - CS * NS

_MESH = plsc.VectorSubcoreMesh(
    core_axis_name="c", subcore_axis_name="s", num_cores=NC, num_subcores=NS
)


def _worker_id():
    return lax.axis_index("s") * NC + lax.axis_index("c")


@functools.partial(
    pl.kernel,
    out_type=jax.ShapeDtypeStruct((N, 16), jnp.float32),
    mesh=_MESH,
    scratch_types=[
        pltpu.VMEM((PW, 1, CHUNK), jnp.int32),
        pltpu.VMEM((1, 1, CHUNK), jnp.int32),
        pltpu.VMEM((CHUNK, 16), jnp.float32),
        pltpu.VMEM((CHUNK, 16), jnp.float32),
        pltpu.SemaphoreType.DMA,
        pltpu.SemaphoreType.DMA,
    ],
    compiler_params=pltpu.CompilerParams(use_tc_tiling_on_sc=False),
)
def _gather_centers(tab, lab, out, idx_v, idx_t, buf0, buf1, sem0, sem1):
    wid = _worker_id()
    base = wid * PW
    pltpu.sync_copy(lab.at[pl.ds(base, PW)], idx_v)

    pltpu.async_copy(tab.at[idx_v.at[0, 0]], buf0, sem0)

    def body(j, carry):
        pltpu.async_copy(tab.at[idx_v.at[2 * j + 1, 0]], buf1, sem1)
        pltpu.make_async_copy(tab.at[idx_v.at[2 * j, 0]], buf0, sem0).wait()
        pltpu.sync_copy(buf0, out.at[pl.ds((base + 2 * j) * CHUNK, CHUNK)])

        @pl.when(j < PW // 2 - 1)
        def _():
            pltpu.async_copy(tab.at[idx_v.at[2 * j + 2, 0]], buf0, sem0)

        pltpu.make_async_copy(tab.at[idx_v.at[2 * j + 1, 0]], buf1, sem1).wait()
        pltpu.sync_copy(buf1, out.at[pl.ds((base + 2 * j + 1) * CHUNK, CHUNK)])
        return carry

    lax.fori_loop(0, PW // 2, body, 0)

    @pl.when(wid < TAIL)
    def _():
        c = NW * PW + wid
        pltpu.sync_copy(lab.at[pl.ds(c, 1)], idx_t)
        pltpu.async_copy(tab.at[idx_t.at[0, 0]], buf0, sem0).wait()
        pltpu.sync_copy(buf0, out.at[pl.ds(c * CHUNK, CHUNK)])


def _attn_body(pf_ref, cp_ref, pt_ref, w1a_ref, w1b_ref, w3a_ref, w3b_ref,
               b1_ref, w2rep_ref, b2_ref, z_ref):
    pf = pf_ref[...]
    dT = cp_ref[...] - pt_ref[...]
    hd = lax.dot_general(dT, w1b_ref[...], (((0,), (0,)), ((), ())),
                         preferred_element_type=jnp.float32)
    gzd = lax.dot_general(dT, w3b_ref[...], (((0,), (0,)), ((), ())),
                          preferred_element_type=jnp.float32)
    h = jnp.maximum(
        jnp.dot(pf, w1a_ref[...], preferred_element_type=jnp.float32)
        + hd + b1_ref[...],
        0.0,
    )
    gz = (
        jnp.dot(pf, w3a_ref[...], preferred_element_type=jnp.float32)
        + gzd
    )
    logit = jnp.dot(h, w2rep_ref[...], preferred_element_type=jnp.float32) + b2_ref[...]
    a = 1.0 / (1.0 + jnp.exp(-logit))
    z_ref[...] = gz * a


def _attention(pf, cpT, ptT, w1a, w1b16, w3a, w3b16, b1r, w2rep, b2r, bn):
    grid = (N // bn,)
    return pl.pallas_call(
        _attn_body,
        grid=grid,
        in_specs=[
            pl.BlockSpec((bn, DF), lambda i: (i, 0)),
            pl.BlockSpec((16, bn), lambda i: (0, i)),
            pl.BlockSpec((16, bn), lambda i: (0, i)),
            pl.BlockSpec((DF, 64), lambda i: (0, 0)),
            pl.BlockSpec((16, 64), lambda i: (0, 0)),
            pl.BlockSpec((DF, 128), lambda i: (0, 0)),
            pl.BlockSpec((16, 128), lambda i: (0, 0)),
            pl.BlockSpec((1, 64), lambda i: (0, 0)),
            pl.BlockSpec((64, 128), lambda i: (0, 0)),
            pl.BlockSpec((1, 1), lambda i: (0, 0)),
        ],
        out_specs=pl.BlockSpec((bn, DZ), lambda i: (i, 0)),
        out_shape=jax.ShapeDtypeStruct((N, DZ), jnp.float32),
    )(pf, cpT, ptT, w1a, w1b16, w3a, w3b16, b1r, w2rep, b2r)


@functools.partial(
    pl.kernel,
    out_type=jax.ShapeDtypeStruct((NC, C, DZ), jnp.float32),
    mesh=_MESH,
    scratch_types=[
        pltpu.VMEM((PW, 1, CHUNK), jnp.int32),
        pltpu.VMEM((1, 1, CHUNK), jnp.int32),
        pltpu.VMEM((CHUNK, DZ), jnp.float32),
        pltpu.VMEM((CHUNK, DZ), jnp.float32),
        pltpu.VMEM_SHARED((C, DZ), jnp.float32),
        pltpu.SemaphoreType.DMA,
        pltpu.SemaphoreType.DMA,
    ],
    compiler_params=pltpu.CompilerParams(use_tc_tiling_on_sc=True),
)
def _segment_sum(lab, z, zeros, p_out, idx_v, idx_t, buf0, buf1, table, sem0, sem1):
    cid = lax.axis_index("c")
    sid = lax.axis_index("s")
    wid = sid * NC + cid
    base = wid * PW
    pltpu.sync_copy(zeros.at[pl.ds(sid * CS, CS)], table.at[pl.ds(sid * CS, CS)])

    @pl.when(sid == 0)
    def _():
        pltpu.sync_copy(
            zeros.at[pl.ds(NS * CS, CSREM)], table.at[pl.ds(NS * CS, CSREM)]
        )

    plsc.subcore_barrier()

    pltpu.sync_copy(lab.at[pl.ds(base, PW)], idx_v)

    pltpu.async_copy(z.at[pl.ds(base * CHUNK, CHUNK)], buf0, sem0)

    def body(j, carry):
        pltpu.async_copy(z.at[pl.ds((base + 2 * j + 1) * CHUNK, CHUNK)], buf1, sem1)
        pltpu.make_async_copy(z.at[pl.ds((base + 2 * j) * CHUNK, CHUNK)], buf0, sem0).wait()
        pltpu.sync_copy(buf0, table.at[idx_v.at[2 * j, 0]], add=True)

        @pl.when(j < PW // 2 - 1)
        def _():
            pltpu.async_copy(z.at[pl.ds((base + 2 * j + 2) * CHUNK, CHUNK)], buf0, sem0)

        pltpu.make_async_copy(z.at[pl.ds((base + 2 * j + 1) * CHUNK, CHUNK)], buf1, sem1).wait()
        pltpu.sync_copy(buf1, table.at[idx_v.at[2 * j + 1, 0]], add=True)
        return carry

    lax.fori_loop(0, PW // 2, body, 0)

    @pl.when(wid < TAIL)
    def _():
        c = NW * PW + wid
        pltpu.sync_copy(lab.at[pl.ds(c, 1)], idx_t)
        pltpu.sync_copy(z.at[pl.ds(c * CHUNK, CHUNK)], buf0)
        pltpu.sync_copy(buf0, table.at[idx_t.at[0, 0]], add=True)

    plsc.subcore_barrier()
    pltpu.sync_copy(
        table.at[pl.ds(sid * CS, CS)], p_out.at[cid, pl.ds(sid * CS, CS)]
    )

    @pl.when(sid == 0)
    def _():
        pltpu.sync_copy(
            table.at[pl.ds(NS * CS, CSREM)], p_out.at[cid, pl.ds(NS * CS, CSREM)]
        )


def _mlp_body(p_ref, b3_ref, w4_ref, b4_ref, out_ref):
    h = jnp.maximum(p_ref[0] + p_ref[1] + b3_ref[...], 0.0)
    out = jnp.dot(h, w4_ref[...], preferred_element_type=jnp.float32) + b4_ref[...]
    out_ref[...] = jnp.maximum(out, 0.0)


def _output_mlp(partials, b3r, w4, b4r, bc):
    grid = (C // bc,)
    return pl.pallas_call(
        _mlp_body,
        grid=grid,
        in_specs=[
            pl.BlockSpec((NC, bc, DZ), lambda i: (0, i, 0)),
            pl.BlockSpec((1, 128), lambda i: (0, 0)),
            pl.BlockSpec((128, 256), lambda i: (0, 0)),
            pl.BlockSpec((1, 256), lambda i: (0, 0)),
        ],
        out_specs=pl.BlockSpec((bc, 256), lambda i: (i, 0)),
        out_shape=jax.ShapeDtypeStruct((C, 256), jnp.float32),
    )(partials, b3r, w4, b4r)


def kernel(point_features, labels, cluster_centers, points, W1, b1, W2, b2, W3, b3, W4, b4):
    lab2d = labels.astype(jnp.int32).reshape(NCHUNK, 1, CHUNK)
    cent16 = jnp.pad(cluster_centers, ((0, 0), (0, 13)))

    cp16 = _gather_centers(cent16, lab2d)
    cpT = cp16.T
    ptT = jnp.pad(points.T, ((0, 13), (0, 0)))

    w1a = W1[:DF]
    w1b16 = jnp.pad(W1[DF:], ((0, 13), (0, 0)))
    w3a = W3[:DF]
    w3b16 = jnp.pad(W3[DF:], ((0, 13), (0, 0)))
    w2rep = jnp.tile(W2, (1, 128))
    z = _attention(
        point_features, cpT, ptT,
        w1a, w1b16, w3a, w3b16, b1.reshape(1, 64), w2rep, b2.reshape(1, 1),
        bn=16000,
    )

    zeros = jnp.zeros((C, DZ), jnp.float32)
    partials = _segment_sum(lab2d, z, zeros)

    return _output_mlp(
        partials, b3.reshape(1, 128), W4, b4.reshape(1, 256), bc=1000
    )


# speedup vs baseline: 1.0308x; 1.0308x over previous
import functools

import jax
import jax.numpy as jnp
from jax import lax
from jax.experimental import pallas as pl
from jax.experimental.pallas import tpu as pltpu
from jax.experimental.pallas import tpu_sc as plsc

N = 320000
C = 10000
DF = 128
DZ = 128
CHUNK = 128
NCHUNK = N // CHUNK
NC, NS = 2, 16
NW = NC * NS
PW = NCHUNK // NW
TAIL = NCHUNK - PW * NW
CS = 624
CSREM = C - CS * NS

_MESH = plsc.VectorSubcoreMesh(
    core_axis_name="c", subcore_axis_name="s", num_cores=NC, num_subcores=NS
)


def _worker_id():
    return lax.axis_index("s") * NC + lax.axis_index("c")


@functools.partial(
    pl.kernel,
    out_type=jax.ShapeDtypeStruct((N, 16), jnp.float32),
    mesh=_MESH,
    scratch_types=[
        pltpu.VMEM((PW, 1, CHUNK), jnp.int32),
        pltpu.VMEM((1, 1, CHUNK), jnp.int32),
        pltpu.VMEM((CHUNK, 16), jnp.float32),
        pltpu.VMEM((CHUNK, 16), jnp.float32),
        pltpu.SemaphoreType.DMA,
        pltpu.SemaphoreType.DMA,
    ],
    compiler_params=pltpu.CompilerParams(use_tc_tiling_on_sc=False),
)
def _gather_centers(tab, lab, out, idx_v, idx_t, buf0, buf1, sem0, sem1):
    wid = _worker_id()
    base = wid * PW
    pltpu.sync_copy(lab.at[pl.ds(base, PW)], idx_v)

    pltpu.async_copy(tab.at[idx_v.at[0, 0]], buf0, sem0)

    def body(j, carry):
        pltpu.async_copy(tab.at[idx_v.at[2 * j + 1, 0]], buf1, sem1)
        pltpu.make_async_copy(tab.at[idx_v.at[2 * j, 0]], buf0, sem0).wait()
        pltpu.sync_copy(buf0, out.at[pl.ds((base + 2 * j) * CHUNK, CHUNK)])

        @pl.when(j < PW // 2 - 1)
        def _():
            pltpu.async_copy(tab.at[idx_v.at[2 * j + 2, 0]], buf0, sem0)

        pltpu.make_async_copy(tab.at[idx_v.at[2 * j + 1, 0]], buf1, sem1).wait()
        pltpu.sync_copy(buf1, out.at[pl.ds((base + 2 * j + 1) * CHUNK, CHUNK)])
        return carry

    lax.fori_loop(0, PW // 2, body, 0)

    @pl.when(wid < TAIL)
    def _():
        c = NW * PW + wid
        pltpu.sync_copy(lab.at[pl.ds(c, 1)], idx_t)
        pltpu.async_copy(tab.at[idx_t.at[0, 0]], buf0, sem0).wait()
        pltpu.sync_copy(buf0, out.at[pl.ds(c * CHUNK, CHUNK)])


def _attn_body(pf_ref, cp_ref, pt_ref, w1a_ref, w1b_ref, w3a_ref, w3b_ref,
               b1_ref, w2rep_ref, b2_ref, z_ref):
    pf = pf_ref[...]
    dT = cp_ref[...] - pt_ref[...]
    hd = lax.dot_general(dT, w1b_ref[...], (((0,), (0,)), ((), ())),
                         preferred_element_type=jnp.float32)
    gzd = lax.dot_general(dT, w3b_ref[...], (((0,), (0,)), ((), ())),
                          preferred_element_type=jnp.float32)
    h = jnp.maximum(
        jnp.dot(pf, w1a_ref[...], preferred_element_type=jnp.float32)
        + hd + b1_ref[...],
        0.0,
    )
    gz = (
        jnp.dot(pf, w3a_ref[...], preferred_element_type=jnp.float32)
        + gzd
    )
    logit = jnp.dot(h, w2rep_ref[...], preferred_element_type=jnp.float32) + b2_ref[...]
    a = 1.0 / (1.0 + jnp.exp(-logit))
    z_ref[...] = gz * a


def _attention(pf, cpT, ptT, w1a, w1b16, w3a, w3b16, b1r, w2rep, b2r, bn,
               row_lo, nrows):
    off = row_lo // bn
    grid = (nrows // bn,)
    return pl.pallas_call(
        _attn_body,
        grid=grid,
        in_specs=[
            pl.BlockSpec((bn, DF), lambda i: (i + off, 0)),
            pl.BlockSpec((16, bn), lambda i: (0, i + off)),
            pl.BlockSpec((16, bn), lambda i: (0, i + off)),
            pl.BlockSpec((DF, 64), lambda i: (0, 0)),
            pl.BlockSpec((16, 64), lambda i: (0, 0)),
            pl.BlockSpec((DF, 128), lambda i: (0, 0)),
            pl.BlockSpec((16, 128), lambda i: (0, 0)),
            pl.BlockSpec((1, 64), lambda i: (0, 0)),
            pl.BlockSpec((64, 128), lambda i: (0, 0)),
            pl.BlockSpec((1, 1), lambda i: (0, 0)),
        ],
        out_specs=pl.BlockSpec((bn, DZ), lambda i: (i, 0)),
        out_shape=jax.ShapeDtypeStruct((nrows, DZ), jnp.float32),
    )(pf, cpT, ptT, w1a, w1b16, w3a, w3b16, b1r, w2rep, b2r)


def _make_segment_sum(chunk_lo, nchunks):
    pw = nchunks // NW
    tail = nchunks - pw * NW
    npairs = pw // 2
    odd = pw - 2 * npairs

    @functools.partial(
        pl.kernel,
        out_type=jax.ShapeDtypeStruct((NC, C, DZ), jnp.float32),
        mesh=_MESH,
        scratch_types=[
            pltpu.VMEM((pw, 1, CHUNK), jnp.int32),
            pltpu.VMEM((1, 1, CHUNK), jnp.int32),
            pltpu.VMEM((CHUNK, DZ), jnp.float32),
            pltpu.VMEM((CHUNK, DZ), jnp.float32),
            pltpu.VMEM_SHARED((C, DZ), jnp.float32),
            pltpu.SemaphoreType.DMA,
            pltpu.SemaphoreType.DMA,
        ],
        compiler_params=pltpu.CompilerParams(use_tc_tiling_on_sc=True),
    )
    def _seg(lab, z, zeros, p_out, idx_v, idx_t, buf0, buf1, table, sem0, sem1):
        cid = lax.axis_index("c")
        sid = lax.axis_index("s")
        wid = sid * NC + cid
        base = wid * pw
        pltpu.sync_copy(zeros.at[pl.ds(sid * CS, CS)], table.at[pl.ds(sid * CS, CS)])

        @pl.when(sid == 0)
        def _():
            pltpu.sync_copy(
                zeros.at[pl.ds(NS * CS, CSREM)], table.at[pl.ds(NS * CS, CSREM)]
            )

        plsc.subcore_barrier()

        pltpu.sync_copy(lab.at[pl.ds(chunk_lo + base, pw)], idx_v)

        pltpu.async_copy(z.at[pl.ds(base * CHUNK, CHUNK)], buf0, sem0)

        def body(j, carry):
            pltpu.async_copy(z.at[pl.ds((base + 2 * j + 1) * CHUNK, CHUNK)], buf1, sem1)
            pltpu.make_async_copy(z.at[pl.ds((base + 2 * j) * CHUNK, CHUNK)], buf0, sem0).wait()
            pltpu.sync_copy(buf0, table.at[idx_v.at[2 * j, 0]], add=True)

            @pl.when(j < npairs - 1 + odd)
            def _():
                pltpu.async_copy(z.at[pl.ds((base + 2 * j + 2) * CHUNK, CHUNK)], buf0, sem0)

            pltpu.make_async_copy(z.at[pl.ds((base + 2 * j + 1) * CHUNK, CHUNK)], buf1, sem1).wait()
            pltpu.sync_copy(buf1, table.at[idx_v.at[2 * j + 1, 0]], add=True)
            return carry

        lax.fori_loop(0, npairs, body, 0)

        if odd:
            pltpu.make_async_copy(
                z.at[pl.ds((base + pw - 1) * CHUNK, CHUNK)], buf0, sem0
            ).wait()
            pltpu.sync_copy(buf0, table.at[idx_v.at[pw - 1, 0]], add=True)

        if tail:
            @pl.when(wid < tail)
            def _():
                c = NW * pw + wid
                pltpu.sync_copy(lab.at[pl.ds(chunk_lo + c, 1)], idx_t)
                pltpu.sync_copy(z.at[pl.ds(c * CHUNK, CHUNK)], buf0)
                pltpu.sync_copy(buf0, table.at[idx_t.at[0, 0]], add=True)

        plsc.subcore_barrier()
        pltpu.sync_copy(
            table.at[pl.ds(sid * CS, CS)], p_out.at[cid, pl.ds(sid * CS, CS)]
        )

        @pl.when(sid == 0)
        def _():
            pltpu.sync_copy(
                table.at[pl.ds(NS * CS, CSREM)], p_out.at[cid, pl.ds(NS * CS, CSREM)]
            )

    return _seg


_seg_half0 = _make_segment_sum(0, NCHUNK // 2)
_seg_half1 = _make_segment_sum(NCHUNK // 2, NCHUNK - NCHUNK // 2)


def _mlp_body(p_ref, q_ref, b3_ref, w4_ref, b4_ref, out_ref):
    h = jnp.maximum(
        p_ref[0] + p_ref[1] + q_ref[0] + q_ref[1] + b3_ref[...], 0.0
    )
    out = jnp.dot(h, w4_ref[...], preferred_element_type=jnp.float32) + b4_ref[...]
    out_ref[...] = jnp.maximum(out, 0.0)


def _output_mlp(partials, partials2, b3r, w4, b4r, bc):
    grid = (C // bc,)
    return pl.pallas_call(
        _mlp_body,
        grid=grid,
        in_specs=[
            pl.BlockSpec((NC, bc, DZ), lambda i: (0, i, 0)),
            pl.BlockSpec((NC, bc, DZ), lambda i: (0, i, 0)),
            pl.BlockSpec((1, 128), lambda i: (0, 0)),
            pl.BlockSpec((128, 256), lambda i: (0, 0)),
            pl.BlockSpec((1, 256), lambda i: (0, 0)),
        ],
        out_specs=pl.BlockSpec((bc, 256), lambda i: (i, 0)),
        out_shape=jax.ShapeDtypeStruct((C, 256), jnp.float32),
    )(partials, partials2, b3r, w4, b4r)


def kernel(point_features, labels, cluster_centers, points, W1, b1, W2, b2, W3, b3, W4, b4):
    lab2d = labels.astype(jnp.int32).reshape(NCHUNK, 1, CHUNK)
    cent16 = jnp.pad(cluster_centers, ((0, 0), (0, 13)))

    cp16 = _gather_centers(cent16, lab2d)
    cpT = cp16.T
    ptT = jnp.pad(points.T, ((0, 13), (0, 0)))

    w1a = W1[:DF]
    w1b16 = jnp.pad(W1[DF:], ((0, 13), (0, 0)))
    w3a = W3[:DF]
    w3b16 = jnp.pad(W3[DF:], ((0, 13), (0, 0)))
    w2rep = jnp.tile(W2, (1, 128))
    half = N // 2
    args = (w1a, w1b16, w3a, w3b16, b1.reshape(1, 64), w2rep, b2.reshape(1, 1))
    z0 = _attention(point_features, cpT, ptT, *args, bn=6400, row_lo=0, nrows=half)
    z1 = _attention(point_features, cpT, ptT, *args, bn=6400, row_lo=half, nrows=half)

    zeros = jnp.zeros((C, DZ), jnp.float32)
    p0 = _seg_half0(lab2d, z0, zeros)
    p1 = _seg_half1(lab2d, z1, zeros)

    return _output_mlp(
        p0, p1, b3.reshape(1, 128), W4, b4.reshape(1, 256), bc=1000
    )


# speedup vs baseline: 1.0364x; 1.0054x over previous
import functools

import jax
import jax.numpy as jnp
from jax import lax
from jax.experimental import pallas as pl
from jax.experimental.pallas import tpu as pltpu
from jax.experimental.pallas import tpu_sc as plsc

N = 320000
C = 10000
DF = 128
DZ = 128
CHUNK = 128
NCHUNK = N // CHUNK
NC, NS = 2, 16
NW = NC * NS
PW = NCHUNK // NW
TAIL = NCHUNK - PW * NW
CS = 624
CSREM = C - CS * NS

_MESH = plsc.VectorSubcoreMesh(
    core_axis_name="c", subcore_axis_name="s", num_cores=NC, num_subcores=NS
)


def _worker_id():
    return lax.axis_index("s") * NC + lax.axis_index("c")


@functools.partial(
    pl.kernel,
    out_type=jax.ShapeDtypeStruct((N, 16), jnp.float32),
    mesh=_MESH,
    scratch_types=[
        pltpu.VMEM((PW, 1, CHUNK), jnp.int32),
        pltpu.VMEM((1, 1, CHUNK), jnp.int32),
        pltpu.VMEM((CHUNK, 16), jnp.float32),
        pltpu.VMEM((CHUNK, 16), jnp.float32),
        pltpu.SemaphoreType.DMA,
        pltpu.SemaphoreType.DMA,
    ],
    compiler_params=pltpu.CompilerParams(use_tc_tiling_on_sc=False),
)
def _gather_centers(tab, lab, out, idx_v, idx_t, buf0, buf1, sem0, sem1):
    wid = _worker_id()
    base = wid * PW
    pltpu.sync_copy(lab.at[pl.ds(base, PW)], idx_v)

    pltpu.async_copy(tab.at[idx_v.at[0, 0]], buf0, sem0)

    def body(j, carry):
        pltpu.async_copy(tab.at[idx_v.at[2 * j + 1, 0]], buf1, sem1)
        pltpu.make_async_copy(tab.at[idx_v.at[2 * j, 0]], buf0, sem0).wait()
        pltpu.sync_copy(buf0, out.at[pl.ds((base + 2 * j) * CHUNK, CHUNK)])

        @pl.when(j < PW // 2 - 1)
        def _():
            pltpu.async_copy(tab.at[idx_v.at[2 * j + 2, 0]], buf0, sem0)

        pltpu.make_async_copy(tab.at[idx_v.at[2 * j + 1, 0]], buf1, sem1).wait()
        pltpu.sync_copy(buf1, out.at[pl.ds((base + 2 * j + 1) * CHUNK, CHUNK)])
        return carry

    lax.fori_loop(0, PW // 2, body, 0)

    @pl.when(wid < TAIL)
    def _():
        c = NW * PW + wid
        pltpu.sync_copy(lab.at[pl.ds(c, 1)], idx_t)
        pltpu.async_copy(tab.at[idx_t.at[0, 0]], buf0, sem0).wait()
        pltpu.sync_copy(buf0, out.at[pl.ds(c * CHUNK, CHUNK)])


def _attn_body(pf_ref, cp_ref, pt_ref, w1a_ref, w1b_ref, w3a_ref, w3b_ref,
               b1_ref, w2rep_ref, b2_ref, z_ref):
    pf = pf_ref[...]
    dT = cp_ref[...] - pt_ref[...]
    hd = lax.dot_general(dT, w1b_ref[...], (((0,), (0,)), ((), ())),
                         preferred_element_type=jnp.float32)
    gzd = lax.dot_general(dT, w3b_ref[...], (((0,), (0,)), ((), ())),
                          preferred_element_type=jnp.float32)
    h = jnp.maximum(
        jnp.dot(pf, w1a_ref[...], preferred_element_type=jnp.float32)
        + hd + b1_ref[...],
        0.0,
    )
    gz = (
        jnp.dot(pf, w3a_ref[...], preferred_element_type=jnp.float32)
        + gzd
    )
    logit = jnp.dot(h, w2rep_ref[...], preferred_element_type=jnp.float32) + b2_ref[...]
    a = 1.0 / (1.0 + jnp.exp(-logit))
    z_ref[...] = gz * a


def _attention(pf, cpT, ptT, w1a, w1b16, w3a, w3b16, b1r, w2rep, b2r, bn,
               row_lo, nrows):
    off = row_lo // bn
    grid = (nrows // bn,)
    return pl.pallas_call(
        _attn_body,
        grid=grid,
        in_specs=[
            pl.BlockSpec((bn, DF), lambda i: (i + off, 0)),
            pl.BlockSpec((16, bn), lambda i: (0, i + off)),
            pl.BlockSpec((16, bn), lambda i: (0, i + off)),
            pl.BlockSpec((DF, 64), lambda i: (0, 0)),
            pl.BlockSpec((16, 64), lambda i: (0, 0)),
            pl.BlockSpec((DF, 128), lambda i: (0, 0)),
            pl.BlockSpec((16, 128), lambda i: (0, 0)),
            pl.BlockSpec((1, 64), lambda i: (0, 0)),
            pl.BlockSpec((64, 128), lambda i: (0, 0)),
            pl.BlockSpec((1, 1), lambda i: (0, 0)),
        ],
        out_specs=pl.BlockSpec((bn, DZ), lambda i: (i, 0)),
        out_shape=jax.ShapeDtypeStruct((nrows, DZ), jnp.float32),
    )(pf, cpT, ptT, w1a, w1b16, w3a, w3b16, b1r, w2rep, b2r)


def _make_segment_sum(chunk_lo, nchunks):
    pw = nchunks // NW
    tail = nchunks - pw * NW
    npairs = pw // 2
    odd = pw - 2 * npairs

    @functools.partial(
        pl.kernel,
        out_type=jax.ShapeDtypeStruct((NC, C, DZ), jnp.float32),
        mesh=_MESH,
        scratch_types=[
            pltpu.VMEM((pw, 1, CHUNK), jnp.int32),
            pltpu.VMEM((1, 1, CHUNK), jnp.int32),
            pltpu.VMEM((CHUNK, DZ), jnp.float32),
            pltpu.VMEM((CHUNK, DZ), jnp.float32),
            pltpu.VMEM_SHARED((C, DZ), jnp.float32),
            pltpu.SemaphoreType.DMA,
            pltpu.SemaphoreType.DMA,
        ],
        compiler_params=pltpu.CompilerParams(use_tc_tiling_on_sc=True),
    )
    def _seg(lab, z, zeros, p_out, idx_v, idx_t, buf0, buf1, table, sem0, sem1):
        cid = lax.axis_index("c")
        sid = lax.axis_index("s")
        wid = sid * NC + cid
        base = wid * pw
        pltpu.sync_copy(zeros.at[pl.ds(sid * CS, CS)], table.at[pl.ds(sid * CS, CS)])

        @pl.when(sid == 0)
        def _():
            pltpu.sync_copy(
                zeros.at[pl.ds(NS * CS, CSREM)], table.at[pl.ds(NS * CS, CSREM)]
            )

        plsc.subcore_barrier()

        pltpu.sync_copy(lab.at[pl.ds(chunk_lo + base, pw)], idx_v)

        pltpu.async_copy(z.at[pl.ds(base * CHUNK, CHUNK)], buf0, sem0)

        def body(j, carry):
            pltpu.async_copy(z.at[pl.ds((base + 2 * j + 1) * CHUNK, CHUNK)], buf1, sem1)
            pltpu.make_async_copy(z.at[pl.ds((base + 2 * j) * CHUNK, CHUNK)], buf0, sem0).wait()
            pltpu.sync_copy(buf0, table.at[idx_v.at[2 * j, 0]], add=True)

            @pl.when(j < npairs - 1 + odd)
            def _():
                pltpu.async_copy(z.at[pl.ds((base + 2 * j + 2) * CHUNK, CHUNK)], buf0, sem0)

            pltpu.make_async_copy(z.at[pl.ds((base + 2 * j + 1) * CHUNK, CHUNK)], buf1, sem1).wait()
            pltpu.sync_copy(buf1, table.at[idx_v.at[2 * j + 1, 0]], add=True)
            return carry

        lax.fori_loop(0, npairs, body, 0)

        if odd:
            pltpu.make_async_copy(
                z.at[pl.ds((base + pw - 1) * CHUNK, CHUNK)], buf0, sem0
            ).wait()
            pltpu.sync_copy(buf0, table.at[idx_v.at[pw - 1, 0]], add=True)

        if tail:
            @pl.when(wid < tail)
            def _():
                c = NW * pw + wid
                pltpu.sync_copy(lab.at[pl.ds(chunk_lo + c, 1)], idx_t)
                pltpu.sync_copy(z.at[pl.ds(c * CHUNK, CHUNK)], buf0)
                pltpu.sync_copy(buf0, table.at[idx_t.at[0, 0]], add=True)

        plsc.subcore_barrier()
        pltpu.sync_copy(
            table.at[pl.ds(sid * CS, CS)], p_out.at[cid, pl.ds(sid * CS, CS)]
        )

        @pl.when(sid == 0)
        def _():
            pltpu.sync_copy(
                table.at[pl.ds(NS * CS, CSREM)], p_out.at[cid, pl.ds(NS * CS, CSREM)]
            )

    return _seg


_seg_half0 = _make_segment_sum(0, NCHUNK // 2)
_seg_half1 = _make_segment_sum(NCHUNK // 2, NCHUNK - NCHUNK // 2)


def _mlp_body(p_ref, q_ref, b3_ref, w4_ref, b4_ref, out_ref):
    h = jnp.maximum(
        p_ref[0] + p_ref[1] + q_ref[0] + q_ref[1] + b3_ref[...], 0.0
    )
    out = jnp.dot(h, w4_ref[...], preferred_element_type=jnp.float32) + b4_ref[...]
    out_ref[...] = jnp.maximum(out, 0.0)


def _output_mlp(partials, partials2, b3r, w4, b4r, bc):
    grid = (C // bc,)
    return pl.pallas_call(
        _mlp_body,
        grid=grid,
        in_specs=[
            pl.BlockSpec((NC, bc, DZ), lambda i: (0, i, 0)),
            pl.BlockSpec((NC, bc, DZ), lambda i: (0, i, 0)),
            pl.BlockSpec((1, 128), lambda i: (0, 0)),
            pl.BlockSpec((128, 256), lambda i: (0, 0)),
            pl.BlockSpec((1, 256), lambda i: (0, 0)),
        ],
        out_specs=pl.BlockSpec((bc, 256), lambda i: (i, 0)),
        out_shape=jax.ShapeDtypeStruct((C, 256), jnp.float32),
    )(partials, partials2, b3r, w4, b4r)


def kernel(point_features, labels, cluster_centers, points, W1, b1, W2, b2, W3, b3, W4, b4):
    lab2d = labels.astype(jnp.int32).reshape(NCHUNK, 1, CHUNK)
    cent16 = jnp.pad(cluster_centers, ((0, 0), (0, 13)))

    cp16 = _gather_centers(cent16, lab2d)
    cpT = cp16.T
    ptT = jnp.pad(points.T, ((0, 13), (0, 0)))

    w1a = W1[:DF]
    w1b16 = jnp.pad(W1[DF:], ((0, 13), (0, 0)))
    w3a = W3[:DF]
    w3b16 = jnp.pad(W3[DF:], ((0, 13), (0, 0)))
    w2rep = jnp.tile(W2, (1, 128))
    half = N // 2
    args = (w1a, w1b16, w3a, w3b16, b1.reshape(1, 64), w2rep, b2.reshape(1, 1))
    z0 = _attention(point_features, cpT, ptT, *args, bn=16000, row_lo=0, nrows=half)
    z1 = _attention(point_features, cpT, ptT, *args, bn=16000, row_lo=half, nrows=half)

    zeros = jnp.zeros((C, DZ), jnp.float32)
    p0 = _seg_half0(lab2d, z0, zeros)
    p1 = _seg_half1(lab2d, z1, zeros)

    return _output_mlp(
        p0, p1, b3.reshape(1, 128), W4, b4.reshape(1, 256), bc=1000
    )
